# Initial kernel scaffold; baseline (speedup 1.0000x reference)
import jax
import jax.numpy as jnp
from jax.experimental import pallas as pl


def kernel(x, table):
    raise NotImplementedError("write your pallas kernel here")


# speedup vs baseline: 20.4221x; 20.4221x over previous
import functools

import jax
import jax.numpy as jnp
import numpy as np
from jax import lax
from jax.experimental import pallas as pl
from jax.experimental.pallas import tpu as pltpu
from jax.experimental.pallas import tpu_sc as plsc

L = 16
F = 2
T = 2 ** 19
N_MIN = 16
N_MAX = 2048
N_PTS = 262144
B_SCALE = float(np.exp((np.log(float(N_MAX)) - np.log(float(N_MIN))) / (L - 1)))
P1 = np.uint32(2654435761)
P2 = np.uint32(805459861)

RES = np.array([np.floor(N_MIN * (B_SCALE ** l)) for l in range(L)], dtype=np.float32)

NC = 2
NS = 16
NW = NC * NS
PTS_PER_TILE = N_PTS // NW
C = 2048
NCH = PTS_PER_TILE // C
G16 = C // 16


def _tec_body(xt_hbm, tabf_hbm, out_hbm,
              xv, idxa, idxb, feats_a, feats_b, wts, outv, sema, semb):
    wid = lax.axis_index("s") * NC + lax.axis_index("c")

    def chunk_body(ch, _):
        base = wid * PTS_PER_TILE + ch * C
        for d in range(3):
            pltpu.sync_copy(xt_hbm.at[pl.ds(d * N_PTS + base, C)],
                            xv.at[pl.ds(d * C, C)])

        for l in range(L):
            res = float(RES[l])
            toff = l * (2 * T)

            def idx_body(g, _):
                p = g * 16
                xs0 = xv[pl.ds(p, 16)] * res
                xs1 = xv[pl.ds(C + p, 16)] * res
                xs2 = xv[pl.ds(2 * C + p, 16)] * res
                i0 = xs0.astype(jnp.int32)
                i1 = xs1.astype(jnp.int32)
                i2 = xs2.astype(jnp.int32)
                w0 = xs0 - i0.astype(jnp.float32)
                w1 = xs1 - i1.astype(jnp.float32)
                w2 = xs2 - i2.astype(jnp.float32)
                v0 = 1.0 - w0
                v1 = 1.0 - w1
                v2 = 1.0 - w2
                u0 = i0.astype(jnp.uint32)
                a0 = u0
                a0b = u0 + jnp.uint32(1)
                a1 = i1.astype(jnp.uint32) * P1
                a1b = a1 + P1
                a2 = i2.astype(jnp.uint32) * P2
                a2b = a2 + P2
                for k in range(8):
                    h = ((a0b if (k >> 2) & 1 else a0)
                         ^ (a1b if (k >> 1) & 1 else a1)
                         ^ (a2b if k & 1 else a2))
                    hm = (h & jnp.uint32(T - 1)).astype(jnp.int32)
                    ia = hm * 2 + toff
                    sl = pl.ds(k * C + p, 16)
                    idxa[sl] = ia
                    idxb[sl] = ia + 1
                    wk = ((w0 if (k >> 2) & 1 else v0)
                          * (w1 if (k >> 1) & 1 else v1)
                          * (w2 if k & 1 else v2))
                    wts[sl] = wk
                return 0

            lax.fori_loop(0, G16, idx_body, 0)

            cpa = pltpu.async_copy(tabf_hbm.at[idxa], feats_a, sema)
            cpb = pltpu.async_copy(tabf_hbm.at[idxb], feats_b, semb)
            cpa.wait()
            cpb.wait()

            def fma_body(g, _):
                p = g * 16
                acc0 = jnp.zeros((16,), jnp.float32)
                acc1 = jnp.zeros((16,), jnp.float32)
                for k in range(8):
                    sl = pl.ds(k * C + p, 16)
                    wk = wts[sl]
                    acc0 = acc0 + wk * feats_a[sl]
                    acc1 = acc1 + wk * feats_b[sl]
                outv[pl.ds(p, 16)] = acc0
                outv[pl.ds(C + p, 16)] = acc1
                return 0

            lax.fori_loop(0, G16, fma_body, 0)

            pltpu.sync_copy(outv.at[pl.ds(0, C)],
                            out_hbm.at[pl.ds((2 * l) * N_PTS + base, C)])
            pltpu.sync_copy(outv.at[pl.ds(C, C)],
                            out_hbm.at[pl.ds((2 * l + 1) * N_PTS + base, C)])
        return 0

    lax.fori_loop(0, NCH, chunk_body, 0)


@jax.jit
def _encode(xt, tabf):
    mesh = plsc.VectorSubcoreMesh(core_axis_name="c", subcore_axis_name="s")
    k = pl.kernel(
        _tec_body,
        out_type=jax.ShapeDtypeStruct((L * F * N_PTS,), jnp.float32),
        mesh=mesh,
        scratch_types=[
            pltpu.VMEM((3 * C,), jnp.float32),
            pltpu.VMEM((8 * C,), jnp.int32),
            pltpu.VMEM((8 * C,), jnp.int32),
            pltpu.VMEM((8 * C,), jnp.float32),
            pltpu.VMEM((8 * C,), jnp.float32),
            pltpu.VMEM((8 * C,), jnp.float32),
            pltpu.VMEM((F * C,), jnp.float32),
            pltpu.SemaphoreType.DMA,
            pltpu.SemaphoreType.DMA,
        ],
    )
    out = k(xt, tabf)
    return jnp.transpose(out.reshape(L * F, N_PTS), (1, 0))


def kernel(x, table):
    xt = x.T.reshape(3 * N_PTS)
    tabf = table.reshape(L * T * F)
    return _encode(xt, tabf)
